# baseline (speedup 1.0000x reference)
import functools

import jax
import jax.numpy as jnp
from jax import lax
from jax.experimental import pallas as pl
from jax.experimental.pallas import tpu as pltpu
from jax.experimental.pallas import tpu_sc as plsc

N_NODES = 10000
N_EDGES = 320000
D_FEAT = 128
D_EDGE = 16
D_HALF = D_FEAT // 2

NC = 2
NS = 16
CHUNK = 128
E_TILE = N_EDGES // NS
IBLK = 10
NBLK = 16
E_TILE_PAD = NBLK * IBLK * CHUNK
PAD_DST = N_NODES + 100
N_PAD = 10240
ROWS_PER_TILE = N_PAD // NS


def _sc_segment_sums(xsplit, src4, dst4, ea4, z64, z16, ones16):
    mesh = plsc.VectorSubcoreMesh(core_axis_name="c", subcore_axis_name="s")

    @functools.partial(
        pl.kernel,
        out_type=[
            jax.ShapeDtypeStruct((NC, N_PAD, D_HALF), jnp.float32),
            jax.ShapeDtypeStruct((NC, N_PAD, D_EDGE), jnp.float32),
            jax.ShapeDtypeStruct((NC, N_PAD, D_EDGE), jnp.float32),
        ],
        mesh=mesh,
        compiler_params=pltpu.CompilerParams(use_tc_tiling_on_sc=False),
        scratch_types=[
            pltpu.VMEM((IBLK, CHUNK), jnp.int32),
            pltpu.VMEM((IBLK, CHUNK), jnp.int32),
            pltpu.VMEM((CHUNK, D_HALF), jnp.float32),
            pltpu.VMEM((CHUNK, D_HALF), jnp.float32),
            pltpu.VMEM((CHUNK, D_EDGE), jnp.float32),
            pltpu.VMEM((CHUNK, D_EDGE), jnp.float32),
            pltpu.VMEM_SHARED((N_PAD, D_HALF), jnp.float32),
            pltpu.VMEM_SHARED((N_PAD, D_EDGE), jnp.float32),
            pltpu.VMEM_SHARED((N_PAD, D_EDGE), jnp.float32),
            pltpu.SemaphoreType.DMA,
            pltpu.SemaphoreType.DMA,
        ],
    )
    def k(x_hbm, src_hbm, dst_hbm, ea_hbm, z64_hbm, z16_hbm, ones_hbm,
          px_hbm, pe_hbm, pd_hbm,
          src_v, dst_v, rows0, rows1, e_v, ones_v, acc_x, acc_e, acc_d,
          sem0, sem1):
        c = lax.axis_index("c")
        s = lax.axis_index("s")
        base = s * ROWS_PER_TILE
        pltpu.sync_copy(z64_hbm.at[pl.ds(base, ROWS_PER_TILE)],
                        acc_x.at[pl.ds(base, ROWS_PER_TILE)])
        pltpu.sync_copy(z16_hbm.at[pl.ds(base, ROWS_PER_TILE)],
                        acc_e.at[pl.ds(base, ROWS_PER_TILE)])
        pltpu.sync_copy(z16_hbm.at[pl.ds(base, ROWS_PER_TILE)],
                        acc_d.at[pl.ds(base, ROWS_PER_TILE)])
        pltpu.sync_copy(ones_hbm, ones_v)
        plsc.subcore_barrier()

        bufs = (rows0, rows1)
        sems = (sem0, sem1)

        def small_ops(j):
            par = j % 2

            @pl.when(par == c)
            def _():
                pltpu.sync_copy(e_v, acc_e.at[dst_v.at[j]], add=True)

            @pl.when(par != c)
            def _():
                pltpu.sync_copy(ones_v, acc_d.at[dst_v.at[j]], add=True)

        @pl.loop(0, NBLK)
        def _(ob):
            pltpu.sync_copy(src_hbm.at[c, s, ob], src_v)
            pltpu.sync_copy(dst_hbm.at[s, ob], dst_v)
            @pl.loop(0, IBLK)
            def _(jj):
                pltpu.async_copy(x_hbm.at[src_v.at[jj]], rows0, sem0).wait()

                @pl.when((jj % 2) == c)
                def _():
                    pltpu.sync_copy(ea_hbm.at[s, ob * IBLK + jj], e_v)

                pltpu.sync_copy(rows0, acc_x.at[dst_v.at[jj]], add=True)
                small_ops(jj)

        plsc.subcore_barrier()
        pltpu.sync_copy(acc_x.at[pl.ds(base, ROWS_PER_TILE)],
                        px_hbm.at[c, pl.ds(base, ROWS_PER_TILE)])
        pltpu.sync_copy(acc_e.at[pl.ds(base, ROWS_PER_TILE)],
                        pe_hbm.at[c, pl.ds(base, ROWS_PER_TILE)])
        pltpu.sync_copy(acc_d.at[pl.ds(base, ROWS_PER_TILE)],
                        pd_hbm.at[c, pl.ds(base, ROWS_PER_TILE)])

    return k(xsplit, src4, dst4, ea4, z64, z16, ones16)


def _tc_body(x_ref, px_ref, pe_ref, pd_ref, ws_ref, we_ref, wf_ref, b_ref,
             o_ref):
    deg = pd_ref[0, :, 0:1] + pd_ref[1, :, 0:1]
    pe = pe_ref[0] + pe_ref[1]
    agg = (jnp.dot(px_ref[0], ws_ref[0:D_HALF, :],
                   preferred_element_type=jnp.float32)
           + jnp.dot(px_ref[1], ws_ref[D_HALF:D_FEAT, :],
                     preferred_element_type=jnp.float32)
           + jnp.dot(pe, we_ref[...], preferred_element_type=jnp.float32))
    agg = agg / jnp.maximum(deg, 1.0)
    h = jnp.dot(x_ref[...], wf_ref[...], preferred_element_type=jnp.float32)
    o_ref[...] = jnp.maximum(h + agg + b_ref[...], 0.0)


def kernel(x, edge_index, edge_attr, W_src, W_edge, W_self, b):
    src = edge_index[0]
    dst = edge_index[1]
    npad = E_TILE_PAD - E_TILE
    src_p = jnp.concatenate(
        [src.reshape(NS, E_TILE),
         jnp.zeros((NS, npad), jnp.int32)], axis=1)
    dst_p = jnp.concatenate(
        [dst.reshape(NS, E_TILE),
         jnp.full((NS, npad), PAD_DST, jnp.int32)], axis=1)
    ea_p = jnp.concatenate(
        [edge_attr.reshape(NS, E_TILE, D_EDGE),
         jnp.zeros((NS, npad, D_EDGE), jnp.float32)], axis=1)
    src4 = (src_p[None] + jnp.array([0, N_NODES], jnp.int32)[:, None, None]
            ).reshape(NC, NS, NBLK, IBLK, CHUNK)
    dst4 = dst_p.reshape(NS, NBLK, IBLK, CHUNK)
    ea4 = ea_p.reshape(NS, NBLK * IBLK, CHUNK, D_EDGE)
    xsplit = jnp.concatenate([x[:, :D_HALF], x[:, D_HALF:]], axis=0)
    z64 = jnp.zeros((N_PAD, D_HALF), jnp.float32)
    z16 = jnp.zeros((N_PAD, D_EDGE), jnp.float32)
    ones16 = jnp.ones((CHUNK, D_EDGE), jnp.float32)

    px, pe, pd = _sc_segment_sums(xsplit, src4, dst4, ea4, z64, z16, ones16)

    R = 1000
    grid = (N_NODES // R,)
    out = pl.pallas_call(
        _tc_body,
        grid=grid,
        in_specs=[
            pl.BlockSpec((R, D_FEAT), lambda i: (i, 0)),
            pl.BlockSpec((NC, R, D_HALF), lambda i: (0, i, 0)),
            pl.BlockSpec((NC, R, D_EDGE), lambda i: (0, i, 0)),
            pl.BlockSpec((NC, R, D_EDGE), lambda i: (0, i, 0)),
            pl.BlockSpec((D_FEAT, D_FEAT), lambda i: (0, 0)),
            pl.BlockSpec((D_EDGE, D_FEAT), lambda i: (0, 0)),
            pl.BlockSpec((D_FEAT, D_FEAT), lambda i: (0, 0)),
            pl.BlockSpec((1, D_FEAT), lambda i: (0, 0)),
        ],
        out_specs=pl.BlockSpec((R, D_FEAT), lambda i: (i, 0)),
        out_shape=jax.ShapeDtypeStruct((N_NODES, D_FEAT), jnp.float32),
    )(x, px, pe, pd, W_src, W_edge, W_self, b.reshape(1, D_FEAT))
    return out


# speedup vs baseline: 1.5260x; 1.5260x over previous
import functools

import jax
import jax.numpy as jnp
from jax import lax
from jax.experimental import pallas as pl
from jax.experimental.pallas import tpu as pltpu
from jax.experimental.pallas import tpu_sc as plsc

N_NODES = 10000
N_EDGES = 320000
D_FEAT = 128
D_EDGE = 16
D_HALF = D_FEAT // 2

NC = 2
NS = 16
CHUNK = 80
NCHUNK = N_EDGES // (NS * CHUNK)
IBLK = 10
NBLK = NCHUNK // IBLK
N_PAD = 10240
ROWS_PER_TILE = N_PAD // NS


def _sc_segment_sums(xsplit, src4, dst4, ea4, z64, z16, ones16):
    mesh = plsc.VectorSubcoreMesh(core_axis_name="c", subcore_axis_name="s")

    @functools.partial(
        pl.kernel,
        out_type=[
            jax.ShapeDtypeStruct((NC, N_PAD, D_HALF), jnp.float32),
            jax.ShapeDtypeStruct((NC, N_PAD, D_EDGE), jnp.float32),
            jax.ShapeDtypeStruct((NC, N_PAD, D_EDGE), jnp.float32),
        ],
        mesh=mesh,
        compiler_params=pltpu.CompilerParams(use_tc_tiling_on_sc=False),
        scratch_types=[
            pltpu.VMEM((IBLK, CHUNK), jnp.int32),
            pltpu.VMEM((IBLK, CHUNK), jnp.int32),
            pltpu.VMEM((CHUNK, D_HALF), jnp.float32),
            pltpu.VMEM((CHUNK, D_HALF), jnp.float32),
            pltpu.VMEM((CHUNK, D_EDGE), jnp.float32),
            pltpu.VMEM((CHUNK, D_EDGE), jnp.float32),
            pltpu.VMEM_SHARED((N_PAD, D_HALF), jnp.float32),
            pltpu.VMEM_SHARED((N_PAD, D_EDGE), jnp.float32),
            pltpu.VMEM_SHARED((N_PAD, D_EDGE), jnp.float32),
            pltpu.SemaphoreType.DMA,
            pltpu.SemaphoreType.DMA,
        ],
    )
    def k(x_hbm, src_hbm, dst_hbm, ea_hbm, z64_hbm, z16_hbm, ones_hbm,
          px_hbm, pe_hbm, pd_hbm,
          src_v, dst_v, rows0, rows1, e_v, ones_v, acc_x, acc_e, acc_d,
          sem0, sem1):
        c = lax.axis_index("c")
        s = lax.axis_index("s")
        base = s * ROWS_PER_TILE
        pltpu.sync_copy(z64_hbm.at[pl.ds(base, ROWS_PER_TILE)],
                        acc_x.at[pl.ds(base, ROWS_PER_TILE)])
        pltpu.sync_copy(z16_hbm.at[pl.ds(base, ROWS_PER_TILE)],
                        acc_e.at[pl.ds(base, ROWS_PER_TILE)])
        pltpu.sync_copy(z16_hbm.at[pl.ds(base, ROWS_PER_TILE)],
                        acc_d.at[pl.ds(base, ROWS_PER_TILE)])
        pltpu.sync_copy(ones_hbm, ones_v)
        plsc.subcore_barrier()

        bufs = (rows0, rows1)
        sems = (sem0, sem1)

        @pl.loop(0, NBLK)
        def _(ob):
            pltpu.sync_copy(src_hbm.at[c, s, ob], src_v)
            pltpu.sync_copy(dst_hbm.at[s, ob], dst_v)
            pltpu.async_copy(x_hbm.at[src_v.at[0]], rows0, sem0)

            @pl.loop(0, IBLK, step=2)
            def _(j):
                for h in range(2):
                    jj = j + h
                    buf, sem = bufs[h], sems[h]

                    pltpu.make_async_copy(x_hbm.at[src_v.at[jj]],
                                          buf, sem).wait()

                    @pl.when(jj + 1 < IBLK)
                    def _():
                        pltpu.async_copy(x_hbm.at[src_v.at[jj + 1]],
                                         bufs[1 - h], sems[1 - h])

                    pltpu.sync_copy(buf, acc_x.at[dst_v.at[jj]], add=True)

                    @pl.when((jj % 2) == c)
                    def _():
                        pltpu.sync_copy(ea_hbm.at[s, ob * IBLK + jj], e_v)
                        pltpu.sync_copy(e_v, acc_e.at[dst_v.at[jj]],
                                        add=True)

                    @pl.when((jj % 2) != c)
                    def _():
                        pltpu.sync_copy(ones_v, acc_d.at[dst_v.at[jj]],
                                        add=True)

        plsc.subcore_barrier()
        pltpu.sync_copy(acc_x.at[pl.ds(base, ROWS_PER_TILE)],
                        px_hbm.at[c, pl.ds(base, ROWS_PER_TILE)])
        pltpu.sync_copy(acc_e.at[pl.ds(base, ROWS_PER_TILE)],
                        pe_hbm.at[c, pl.ds(base, ROWS_PER_TILE)])
        pltpu.sync_copy(acc_d.at[pl.ds(base, ROWS_PER_TILE)],
                        pd_hbm.at[c, pl.ds(base, ROWS_PER_TILE)])

    return k(xsplit, src4, dst4, ea4, z64, z16, ones16)


def _tc_body(x_ref, px_ref, pe_ref, pd_ref, ws_ref, we_ref, wf_ref, b_ref,
             o_ref):
    deg = pd_ref[0, :, 0:1] + pd_ref[1, :, 0:1]
    pe = pe_ref[0] + pe_ref[1]
    agg = (jnp.dot(px_ref[0], ws_ref[0:D_HALF, :],
                   preferred_element_type=jnp.float32)
           + jnp.dot(px_ref[1], ws_ref[D_HALF:D_FEAT, :],
                     preferred_element_type=jnp.float32)
           + jnp.dot(pe, we_ref[...], preferred_element_type=jnp.float32))
    agg = agg / jnp.maximum(deg, 1.0)
    h = jnp.dot(x_ref[...], wf_ref[...], preferred_element_type=jnp.float32)
    o_ref[...] = jnp.maximum(h + agg + b_ref[...], 0.0)


def kernel(x, edge_index, edge_attr, W_src, W_edge, W_self, b):
    src = edge_index[0]
    dst = edge_index[1]
    src4 = jnp.stack([src, src + N_NODES]).reshape(NC, NS, NBLK, IBLK, CHUNK)
    dst4 = dst.reshape(NS, NBLK, IBLK, CHUNK)
    ea4 = edge_attr.reshape(NS, NBLK * IBLK, CHUNK, D_EDGE)
    xsplit = jnp.concatenate([x[:, :D_HALF], x[:, D_HALF:]], axis=0)
    z64 = jnp.zeros((N_PAD, D_HALF), jnp.float32)
    z16 = jnp.zeros((N_PAD, D_EDGE), jnp.float32)
    ones16 = jnp.ones((CHUNK, D_EDGE), jnp.float32)

    px, pe, pd = _sc_segment_sums(xsplit, src4, dst4, ea4, z64, z16, ones16)

    R = 1000
    grid = (N_NODES // R,)
    out = pl.pallas_call(
        _tc_body,
        grid=grid,
        in_specs=[
            pl.BlockSpec((R, D_FEAT), lambda i: (i, 0)),
            pl.BlockSpec((NC, R, D_HALF), lambda i: (0, i, 0)),
            pl.BlockSpec((NC, R, D_EDGE), lambda i: (0, i, 0)),
            pl.BlockSpec((NC, R, D_EDGE), lambda i: (0, i, 0)),
            pl.BlockSpec((D_FEAT, D_FEAT), lambda i: (0, 0)),
            pl.BlockSpec((D_EDGE, D_FEAT), lambda i: (0, 0)),
            pl.BlockSpec((D_FEAT, D_FEAT), lambda i: (0, 0)),
            pl.BlockSpec((1, D_FEAT), lambda i: (0, 0)),
        ],
        out_specs=pl.BlockSpec((R, D_FEAT), lambda i: (i, 0)),
        out_shape=jax.ShapeDtypeStruct((N_NODES, D_FEAT), jnp.float32),
    )(x, px, pe, pd, W_src, W_edge, W_self, b.reshape(1, D_FEAT))
    return out
